# baseline (speedup 1.0000x reference)
import functools

import jax
import jax.numpy as jnp
from jax import lax
from jax.experimental import pallas as pl
from jax.experimental.pallas import tpu as pltpu
from jax.experimental.pallas import tpu_sc as plsc

B = 16384
D_BLK = 64
D_DIR = 32
D_OUT = D_BLK + D_DIR
NC, NS = 2, 16
NW = NC * NS
BPW = B // NW
NCHUNK = 4
CHUNK = BPW // NCHUNK

_mesh = plsc.VectorSubcoreMesh(core_axis_name="c", subcore_axis_name="s")


@functools.partial(
    pl.kernel,
    out_type=jax.ShapeDtypeStruct((B, D_OUT), jnp.float32),
    mesh=_mesh,
    compiler_params=pltpu.CompilerParams(use_tc_tiling_on_sc=False),
    scratch_types=[
        pltpu.VMEM((NCHUNK, CHUNK), jnp.int32),
        pltpu.VMEM((NCHUNK, CHUNK), jnp.int32),
        pltpu.VMEM((BPW, D_BLK), jnp.float32),
        pltpu.VMEM((BPW, D_DIR), jnp.float32),
        pltpu.SemaphoreType.DMA,
        pltpu.SemaphoreType.DMA,
    ],
)
def _encode(dir_idx_hbm, blk_idx_hbm, dir_tab_hbm, blk_tab_hbm, out_hbm,
            dir_idx_v, blk_idx_v, blk_v, dir_v, sem_b, sem_d):
    wid = lax.axis_index("s") * NC + lax.axis_index("c")
    base = wid * BPW
    pltpu.sync_copy(blk_idx_hbm.at[wid], blk_idx_v)
    pltpu.sync_copy(dir_idx_hbm.at[wid], dir_idx_v)
    copies = []
    for j in range(NCHUNK):
        rows = pl.ds(j * CHUNK, CHUNK)
        copies.append(pltpu.async_copy(
            blk_tab_hbm.at[blk_idx_v.at[j]], blk_v.at[rows], sem_b))
        copies.append(pltpu.async_copy(
            dir_tab_hbm.at[dir_idx_v.at[j]], dir_v.at[rows], sem_d))
    for c in copies:
        c.wait()
    pltpu.sync_copy(blk_v, out_hbm.at[pl.ds(base, BPW), pl.ds(0, D_BLK)])
    pltpu.sync_copy(dir_v, out_hbm.at[pl.ds(base, BPW), pl.ds(D_BLK, D_DIR)])


def kernel(direction_batch, block_batch, direction_table, block_table):
    dir_idx = direction_batch.reshape(NW, NCHUNK, CHUNK)
    blk_idx = block_batch.reshape(NW, NCHUNK, CHUNK)
    return _encode(dir_idx, blk_idx, direction_table, block_table)


# speedup vs baseline: 1.0971x; 1.0971x over previous
import functools

import jax
import jax.numpy as jnp
from jax import lax
from jax.experimental import pallas as pl
from jax.experimental.pallas import tpu as pltpu
from jax.experimental.pallas import tpu_sc as plsc

B = 16384
D_BLK = 64
D_DIR = 32
NC, NS = 2, 16
NW = NC * NS
BPW = B // NW
NCHUNK = 4
CHUNK = BPW // NCHUNK
DP = 128

_mesh = plsc.VectorSubcoreMesh(core_axis_name="c", subcore_axis_name="s")


@functools.partial(
    pl.kernel,
    out_type=(jax.ShapeDtypeStruct((B, DP), jnp.float32),
              jax.ShapeDtypeStruct((B, DP), jnp.float32)),
    mesh=_mesh,
    compiler_params=pltpu.CompilerParams(
        use_tc_tiling_on_sc=False, needs_layout_passes=False),
    scratch_types=[
        pltpu.VMEM((NCHUNK, CHUNK), jnp.int32),
        pltpu.VMEM((NCHUNK, CHUNK), jnp.int32),
        pltpu.VMEM((2, CHUNK, DP), jnp.float32),
        pltpu.VMEM((2, CHUNK, DP), jnp.float32),
        pltpu.SemaphoreType.DMA,
        pltpu.SemaphoreType.DMA,
    ],
)
def _encode(dir_idx_hbm, blk_idx_hbm, dir_tab_hbm, blk_tab_hbm,
            blk_out_hbm, dir_out_hbm,
            dir_idx_v, blk_idx_v, blk_v, dir_v, sem_b, sem_d):
    wid = lax.axis_index("s") * NC + lax.axis_index("c")
    base = wid * BPW
    pltpu.sync_copy(blk_idx_hbm.at[wid], blk_idx_v)
    pltpu.sync_copy(dir_idx_hbm.at[wid], dir_idx_v)

    def drain(slot_state):
        j, cb, cd, s = slot_state
        cb.wait()
        cd.wait()
        rows = pl.ds(base + j * CHUNK, CHUNK)
        pltpu.sync_copy(blk_v.at[s], blk_out_hbm.at[rows])
        pltpu.sync_copy(dir_v.at[s], dir_out_hbm.at[rows])

    pend = [None, None]
    for j in range(NCHUNK):
        s = j % 2
        if pend[s] is not None:
            drain(pend[s])
        cb = pltpu.async_copy(
            blk_tab_hbm.at[blk_idx_v.at[j]], blk_v.at[s], sem_b)
        cd = pltpu.async_copy(
            dir_tab_hbm.at[dir_idx_v.at[j]], dir_v.at[s], sem_d)
        pend[s] = (j, cb, cd, s)
    for s in range(2):
        drain(pend[s])


def kernel(direction_batch, block_batch, direction_table, block_table):
    dir_idx = direction_batch.reshape(NW, NCHUNK, CHUNK)
    blk_idx = block_batch.reshape(NW, NCHUNK, CHUNK)
    blk_p = jnp.pad(block_table, ((0, 7), (0, DP - D_BLK)))
    dir_p = jnp.pad(direction_table, ((0, 6), (0, DP - D_DIR)))
    blk_rows, dir_rows = _encode(dir_idx, blk_idx, dir_p, blk_p)
    return jnp.concatenate(
        [blk_rows[:, :D_BLK], dir_rows[:, :D_DIR]], axis=1)


# speedup vs baseline: 1.4580x; 1.3289x over previous
import functools

import jax
import jax.numpy as jnp
from jax import lax
from jax.experimental import pallas as pl
from jax.experimental.pallas import tpu as pltpu
from jax.experimental.pallas import tpu_sc as plsc

B = 16384
D_BLK = 64
D_DIR = 32
NC, NS = 2, 16
NW = NC * NS
BPW = B // NW
NGRP = BPW // 16
NCK = 4
GPC = NGRP // NCK

_mesh = plsc.VectorSubcoreMesh(core_axis_name="c", subcore_axis_name="s")


@functools.partial(
    pl.kernel,
    out_type=jax.ShapeDtypeStruct((B, 128), jnp.float32),
    mesh=_mesh,
    compiler_params=pltpu.CompilerParams(needs_layout_passes=False),
    scratch_types=[
        pltpu.VMEM((BPW,), jnp.int32),
        pltpu.VMEM((BPW,), jnp.int32),
        pltpu.VMEM((16, 8, D_BLK), jnp.float32),
        pltpu.VMEM((16, 8, D_DIR), jnp.float32),
        pltpu.VMEM((BPW // NCK, 128), jnp.float32),
        pltpu.SemaphoreType.DMA,
        pltpu.SemaphoreType.DMA,
    ],
)
def _encode(dir_idx_hbm, blk_idx_hbm, dir_tab_hbm, blk_tab_hbm, out_hbm,
            dir_idx_v, blk_idx_v, oct_v, doct_v, out_c, sem_b, sem_d):
    wid = lax.axis_index("s") * NC + lax.axis_index("c")
    base = wid * BPW
    pltpu.sync_copy(blk_idx_hbm.at[pl.ds(base, BPW)], blk_idx_v)
    pltpu.sync_copy(dir_idx_hbm.at[pl.ds(base, BPW)], dir_idx_v)
    lanes = lax.broadcasted_iota(jnp.int32, (16,), 0)
    zeros = jnp.zeros((16,), jnp.int32)

    def do_group(c, g):
        off = c * (BPW // NCK) + g * 16
        bvec = blk_idx_v[pl.ds(off, 16)]
        dvec = dir_idx_v[pl.ds(off, 16)]
        handles = []
        for l in range(16):
            m = lanes == l
            bi = lax.reduce_max(jnp.where(m, bvec, zeros), axes=(0,))
            di = lax.reduce_max(jnp.where(m, dvec, zeros), axes=(0,))
            bo = bi // 8
            do = di // 8
            br = bi - bo * 8
            dr = di - do * 8
            cb = pltpu.async_copy(
                blk_tab_hbm.at[pl.ds(bo * 8, 8)], oct_v.at[l], sem_b)
            cd = pltpu.async_copy(
                dir_tab_hbm.at[pl.ds(do * 8, 8)], doct_v.at[l], sem_d)
            handles.append((cb, cd, br, dr))
        for l, (cb, cd, br, dr) in enumerate(handles):
            cb.wait()
            cd.wait()
            row = g * 16 + l
            for k in range(4):
                out_c[row, pl.ds(k * 16, 16)] = oct_v[l, br,
                                                      pl.ds(k * 16, 16)]
            for k in range(2):
                out_c[row, pl.ds(D_BLK + k * 16, 16)] = doct_v[
                    l, dr, pl.ds(k * 16, 16)]

    def chunk(c, carry):
        lax.fori_loop(0, GPC, lambda g, cc: (do_group(c, g), cc)[1], 0)
        pltpu.sync_copy(out_c,
                        out_hbm.at[pl.ds(base + c * (BPW // NCK),
                                         BPW // NCK)])
        return carry

    lax.fori_loop(0, NCK, chunk, 0)


def kernel(direction_batch, block_batch, direction_table, block_table):
    dir_idx = direction_batch.reshape(B)
    blk_idx = block_batch.reshape(B)
    out = _encode(dir_idx, blk_idx, direction_table, block_table)
    return out[:, :D_BLK + D_DIR]


# speedup vs baseline: 1.4770x; 1.0130x over previous
import functools

import jax
import jax.numpy as jnp
from jax import lax
from jax.experimental import pallas as pl
from jax.experimental.pallas import tpu as pltpu
from jax.experimental.pallas import tpu_sc as plsc

B = 16384
D_BLK = 64
D_DIR = 32
NC, NS = 2, 16
NW = NC * NS
BPW = B // NW
NGRP = BPW // 16
NCK = 4
GPC = NGRP // NCK
CROWS = BPW // NCK

_mesh = plsc.VectorSubcoreMesh(core_axis_name="c", subcore_axis_name="s")


@functools.partial(
    pl.kernel,
    out_type=jax.ShapeDtypeStruct((B, 128), jnp.float32),
    mesh=_mesh,
    compiler_params=pltpu.CompilerParams(needs_layout_passes=False),
    scratch_types=[
        pltpu.VMEM((BPW,), jnp.int32),
        pltpu.VMEM((2, 16, 8, D_BLK), jnp.float32),
        pltpu.VMEM((2, 16, 8, D_DIR), jnp.float32),
        pltpu.VMEM((CROWS, 128), jnp.float32),
        pltpu.SemaphoreType.DMA,
        pltpu.SemaphoreType.DMA,
    ],
)
def _encode(comb_idx_hbm, dir_tab_hbm, blk_tab_hbm, out_hbm,
            idx_v, oct_v, doct_v, out_c, sem_b, sem_d):
    wid = lax.axis_index("s") * NC + lax.axis_index("c")
    base = wid * BPW
    pltpu.sync_copy(comb_idx_hbm.at[pl.ds(base, BPW)], idx_v)
    lanes = lax.broadcasted_iota(jnp.int32, (16,), 0)
    zeros = jnp.zeros((16,), jnp.int32)

    def scalars(g):
        cvec = idx_v[pl.ds(g * 16, 16)]
        out = []
        for l in range(16):
            s = lax.reduce_max(jnp.where(lanes == l, cvec, zeros), axes=(0,))
            bi = s // 1024
            di = s - bi * 1024
            out.append((bi, di))
        return out

    def issue(g):
        slot = lax.rem(g, 2)
        for l, (bi, di) in enumerate(scalars(g)):
            bo = (bi // 8) * 8
            do = (di // 8) * 8
            pltpu.async_copy(
                blk_tab_hbm.at[pl.ds(bo, 8)], oct_v.at[slot, l], sem_b)
            pltpu.async_copy(
                dir_tab_hbm.at[pl.ds(do, 8)], doct_v.at[slot, l], sem_d)

    def drain_extract(g):
        slot = lax.rem(g, 2)
        for l in range(16):
            pltpu.make_async_copy(
                blk_tab_hbm.at[pl.ds(0, 8)], oct_v.at[slot, l], sem_b).wait()
            pltpu.make_async_copy(
                dir_tab_hbm.at[pl.ds(0, 8)], doct_v.at[slot, l], sem_d).wait()
        row0 = lax.rem(g, GPC) * 16
        for l, (bi, di) in enumerate(scalars(g)):
            br = lax.rem(bi, 8)
            dr = lax.rem(di, 8)
            row = row0 + l
            for k in range(4):
                out_c[row, pl.ds(k * 16, 16)] = oct_v[slot, l, br,
                                                      pl.ds(k * 16, 16)]
            for k in range(2):
                out_c[row, pl.ds(D_BLK + k * 16, 16)] = doct_v[
                    slot, l, dr, pl.ds(k * 16, 16)]

    issue(0)

    def step(g, carry):
        @pl.when(g < NGRP - 1)
        def _():
            issue(g + 1)
        drain_extract(g)

        @pl.when(lax.rem(g, GPC) == GPC - 1)
        def _():
            c = g // GPC
            pltpu.sync_copy(out_c, out_hbm.at[pl.ds(base + c * CROWS, CROWS)])
        return carry

    lax.fori_loop(0, NGRP, step, 0)


def kernel(direction_batch, block_batch, direction_table, block_table):
    comb = block_batch.reshape(B) * 1024 + direction_batch.reshape(B)
    out = _encode(comb, direction_table, block_table)
    return out[:, :D_BLK + D_DIR]


# speedup vs baseline: 1.5754x; 1.0666x over previous
import functools

import jax
import jax.numpy as jnp
from jax import lax
from jax.experimental import pallas as pl
from jax.experimental.pallas import tpu as pltpu
from jax.experimental.pallas import tpu_sc as plsc

B = 16384
D_BLK = 64
D_DIR = 32
NC, NS = 2, 16
NW = NC * NS
BPW = B // NW
NGRP = BPW // 16
NCK = 4
GPC = NGRP // NCK
CROWS = BPW // NCK

_mesh = plsc.VectorSubcoreMesh(core_axis_name="c", subcore_axis_name="s")


@functools.partial(
    pl.kernel,
    out_type=jax.ShapeDtypeStruct((B, 128), jnp.float32),
    mesh=_mesh,
    compiler_params=pltpu.CompilerParams(needs_layout_passes=False),
    scratch_types=[
        pltpu.VMEM((BPW,), jnp.int32),
        pltpu.VMEM((2, 16, 8, D_BLK), jnp.float32),
        pltpu.VMEM((1002 * D_DIR,), jnp.float32),
        pltpu.VMEM((CROWS, 128), jnp.float32),
        pltpu.SemaphoreType.DMA,
        pltpu.SemaphoreType.DMA,
    ],
)
def _encode(comb_idx_hbm, dir_tab_hbm, blk_tab_hbm, out_hbm,
            idx_v, oct_v, dir_all_v, out_c, sem_b, sem_d):
    wid = lax.axis_index("s") * NC + lax.axis_index("c")
    base = wid * BPW
    pltpu.sync_copy(comb_idx_hbm.at[pl.ds(base, BPW)], idx_v)
    pltpu.sync_copy(dir_tab_hbm, dir_all_v)
    lanes = lax.broadcasted_iota(jnp.int32, (16,), 0)
    zeros = jnp.zeros((16,), jnp.int32)

    def scalars(g):
        cvec = idx_v[pl.ds(g * 16, 16)]
        out = []
        for l in range(16):
            s = lax.reduce_max(jnp.where(lanes == l, cvec, zeros), axes=(0,))
            bi = s // 1024
            di = s - bi * 1024
            out.append((bi, di))
        return out

    def issue(g):
        slot = lax.rem(g, 2)
        for l, (bi, di) in enumerate(scalars(g)):
            bo = (bi // 8) * 8
            pltpu.async_copy(
                blk_tab_hbm.at[pl.ds(bo, 8)], oct_v.at[slot, l], sem_b)

    def drain_extract(g):
        slot = lax.rem(g, 2)
        for l in range(16):
            pltpu.make_async_copy(
                blk_tab_hbm.at[pl.ds(0, 8)], oct_v.at[slot, l], sem_b).wait()
        row0 = lax.rem(g, GPC) * 16
        for l, (bi, di) in enumerate(scalars(g)):
            br = lax.rem(bi, 8)
            row = row0 + l
            for k in range(4):
                out_c[row, pl.ds(k * 16, 16)] = oct_v[slot, l, br,
                                                      pl.ds(k * 16, 16)]
            for k in range(2):
                out_c[row, pl.ds(D_BLK + k * 16, 16)] = dir_all_v[
                    pl.ds(di * D_DIR + k * 16, 16)]

    issue(0)

    def step(g, carry):
        @pl.when(g < NGRP - 1)
        def _():
            issue(g + 1)
        drain_extract(g)

        @pl.when(lax.rem(g, GPC) == GPC - 1)
        def _():
            c = g // GPC
            pltpu.sync_copy(out_c, out_hbm.at[pl.ds(base + c * CROWS, CROWS)])
        return carry

    lax.fori_loop(0, NGRP, step, 0)


def kernel(direction_batch, block_batch, direction_table, block_table):
    comb = block_batch.reshape(B) * 1024 + direction_batch.reshape(B)
    out = _encode(comb, direction_table.reshape(-1), block_table)
    return out[:, :D_BLK + D_DIR]


# speedup vs baseline: 1.5760x; 1.0004x over previous
import functools

import jax
import jax.numpy as jnp
from jax import lax
from jax.experimental import pallas as pl
from jax.experimental.pallas import tpu as pltpu
from jax.experimental.pallas import tpu_sc as plsc

B = 16384
D_BLK = 64
D_DIR = 32
NC, NS = 2, 16
NW = NC * NS
BPW = B // NW
NGRP = BPW // 16
NCK = 4
GPC = NGRP // NCK
CROWS = BPW // NCK

_mesh = plsc.VectorSubcoreMesh(core_axis_name="c", subcore_axis_name="s")


@functools.partial(
    pl.kernel,
    out_type=jax.ShapeDtypeStruct((B, 128), jnp.float32),
    mesh=_mesh,
    compiler_params=pltpu.CompilerParams(needs_layout_passes=False),
    scratch_types=[
        pltpu.VMEM((BPW,), jnp.int32),
        pltpu.VMEM((2, 16, 8, D_BLK), jnp.float32),
        pltpu.VMEM((1002 * D_DIR,), jnp.float32),
        pltpu.VMEM((CROWS, 128), jnp.float32),
        pltpu.SemaphoreType.DMA,
        pltpu.SemaphoreType.DMA,
    ],
)
def _encode(comb_idx_hbm, dir_tab_hbm, blk_tab_hbm, out_hbm,
            idx_v, oct_v, dir_all_v, out_c, sem_b, sem_d):
    wid = lax.axis_index("s") * NC + lax.axis_index("c")
    base = wid * BPW
    pltpu.sync_copy(comb_idx_hbm.at[pl.ds(base, BPW)], idx_v)
    pltpu.sync_copy(dir_tab_hbm, dir_all_v)
    lanes = lax.broadcasted_iota(jnp.int32, (16,), 0)
    zeros = jnp.zeros((16,), jnp.int32)

    def scalars(g):
        cvec = idx_v[pl.ds(g * 16, 16)]
        out = []
        for l in range(16):
            s = lax.reduce_max(jnp.where(lanes == l, cvec, zeros), axes=(0,))
            bi = s // 1024
            di = s - bi * 1024
            out.append((bi, di))
        return out

    def issue(g, sc):
        slot = lax.rem(g, 2)
        for l, (bi, di) in enumerate(sc):
            bo = (bi // 8) * 8
            pltpu.async_copy(
                blk_tab_hbm.at[pl.ds(bo, 8)], oct_v.at[slot, l], sem_b)

    def drain_extract(g, sc):
        slot = lax.rem(g, 2)
        for l in range(16):
            pltpu.make_async_copy(
                blk_tab_hbm.at[pl.ds(0, 8)], oct_v.at[slot, l], sem_b).wait()
        row0 = lax.rem(g, GPC) * 16
        for l, (bi, di) in enumerate(sc):
            br = lax.rem(bi, 8)
            row = row0 + l
            for k in range(4):
                out_c[row, pl.ds(k * 16, 16)] = oct_v[slot, l, br,
                                                      pl.ds(k * 16, 16)]
            for k in range(2):
                out_c[row, pl.ds(D_BLK + k * 16, 16)] = dir_all_v[
                    pl.ds(di * D_DIR + k * 16, 16)]

    issue(0, scalars(0))

    def step(g, carry):
        sc_cur = [(carry[2 * l], carry[2 * l + 1]) for l in range(16)]
        sc_next = scalars(lax.rem(g + 1, NGRP))
        @pl.when(g < NGRP - 1)
        def _():
            issue(g + 1, sc_next)
        drain_extract(g, sc_cur)

        @pl.when(lax.rem(g, GPC) == GPC - 1)
        def _():
            c = g // GPC
            pltpu.sync_copy(out_c, out_hbm.at[pl.ds(base + c * CROWS, CROWS)])
        return tuple(x for p in sc_next for x in p)

    lax.fori_loop(0, NGRP, step,
                  tuple(x for p in scalars(0) for x in p))


def kernel(direction_batch, block_batch, direction_table, block_table):
    comb = block_batch.reshape(B) * 1024 + direction_batch.reshape(B)
    out = _encode(comb, direction_table.reshape(-1), block_table)
    return out[:, :D_BLK + D_DIR]
